# baseline (speedup 1.0000x reference)
import functools

import jax
import jax.numpy as jnp
from jax import lax
from jax.experimental import pallas as pl
from jax.experimental.pallas import tpu as pltpu
from jax.experimental.pallas import tpu_sc as plsc

SMALL = 1e-10
EMB = 128
B = 4096
LC = 200
LC1 = 128
LCP = 256
LQP = 128
LQG = 24
LQ = 20
NE = EMB // 16


def _pool_kernel(b_per_w, code_idx, query_idx, code_table, query_table):
    mesh = plsc.VectorSubcoreMesh(core_axis_name="c", subcore_axis_name="s")
    nc = mesh.num_cores
    npair = b_per_w // 2

    @functools.partial(
        pl.kernel,
        out_type=[
            jax.ShapeDtypeStruct((B, EMB), jnp.float32),
            jax.ShapeDtypeStruct((B, EMB), jnp.float32),
        ],
        mesh=mesh,
        compiler_params=pltpu.CompilerParams(use_tc_tiling_on_sc=False,
                                             needs_layout_passes=False),
        scratch_types=[
            pltpu.VMEM((b_per_w, LCP), jnp.int32),
            pltpu.VMEM((b_per_w, LQP), jnp.int32),
            pltpu.VMEM((LC, EMB), jnp.bfloat16),
            pltpu.VMEM((LC, EMB), jnp.bfloat16),
            pltpu.VMEM((b_per_w, EMB), jnp.float32),
            pltpu.VMEM((b_per_w, EMB), jnp.float32),
            pltpu.SemaphoreType.DMA,
            pltpu.SemaphoreType.DMA,
        ],
    )
    def k(cidx_hbm, qidx_hbm, ctab_hbm, qtab_hbm, outc_hbm, outq_hbm,
          idx_c, idx_q, rows0, rows1, outc, outq, sem0, sem1):
        wid = lax.axis_index("s") * nc + lax.axis_index("c")
        base = wid * b_per_w
        pltpu.sync_copy(cidx_hbm.at[pl.ds(base, b_per_w)], idx_c)
        pltpu.sync_copy(qidx_hbm.at[pl.ds(base, b_per_w)], idx_q)

        def cstart(b, buf, sem):
            pltpu.make_async_copy(
                ctab_hbm.at[idx_c.at[b, pl.ds(0, LC1)]],
                buf.at[pl.ds(0, LC1)], sem).start()
            pltpu.make_async_copy(
                ctab_hbm.at[idx_c.at[b, pl.ds(LC1, LC - LC1)]],
                buf.at[pl.ds(LC1, LC - LC1)], sem).start()

        def cwait(buf, sem):
            pltpu.make_async_copy(
                ctab_hbm.at[idx_c.at[0, pl.ds(0, LC1)]],
                buf.at[pl.ds(0, LC1)], sem).wait()
            pltpu.make_async_copy(
                ctab_hbm.at[idx_c.at[0, pl.ds(LC1, LC - LC1)]],
                buf.at[pl.ds(LC1, LC - LC1)], sem).wait()

        def accum(buf, n, out_ref, b):
            def acc1(l, accs):
                out = list(accs)
                for kk in range(NE // 2):
                    x = buf[l, pl.ds(kk * 32, 32)]
                    u0, u1 = plsc.unpack(
                        x, format=plsc.PackFormat.INTERLEAVED,
                        preferred_element_type=jnp.float32)
                    out[2 * kk] = out[2 * kk] + u0
                    out[2 * kk + 1] = out[2 * kk + 1] + u1
                return tuple(out)
            accs = lax.fori_loop(
                0, n, acc1,
                tuple(jnp.zeros((16,), jnp.float32) for _ in range(NE)))
            for e in range(NE):
                out_ref[b, pl.ds(e * 16, 16)] = accs[e]

        cstart(0, rows0, sem0)

        def code_pair(g, _):
            b0 = 2 * g
            cstart(b0 + 1, rows1, sem1)
            cwait(rows0, sem0)
            accum(rows0, LC, outc, b0)

            @pl.when(g < npair - 1)
            def _():
                cstart(b0 + 2, rows0, sem0)
            cwait(rows1, sem1)
            accum(rows1, LC, outc, b0 + 1)
            return 0

        lax.fori_loop(0, npair, code_pair, 0)

        def qstart(b, buf, sem):
            pltpu.make_async_copy(
                qtab_hbm.at[idx_q.at[b, pl.ds(0, LQG)]],
                buf.at[pl.ds(0, LQG)], sem).start()

        def qwait(buf, sem):
            pltpu.make_async_copy(
                qtab_hbm.at[idx_q.at[0, pl.ds(0, LQG)]],
                buf.at[pl.ds(0, LQG)], sem).wait()

        qstart(0, rows0, sem0)

        def query_pair(g, _):
            b0 = 2 * g
            qstart(b0 + 1, rows1, sem1)
            qwait(rows0, sem0)
            accum(rows0, LQG, outq, b0)

            @pl.when(g < npair - 1)
            def _():
                qstart(b0 + 2, rows0, sem0)
            qwait(rows1, sem1)
            accum(rows1, LQG, outq, b0 + 1)
            return 0

        lax.fori_loop(0, npair, query_pair, 0)

        pltpu.sync_copy(outc, outc_hbm.at[pl.ds(base, b_per_w)])
        pltpu.sync_copy(outq, outq_hbm.at[pl.ds(base, b_per_w)])

    return k(code_idx, query_idx, code_table, query_table)


def _convert_body(x_ref, o_ref):
    o_ref[...] = x_ref[...].astype(jnp.bfloat16)


def _to_bf16(table):
    rows = table.shape[0]
    blk = 5000
    return pl.pallas_call(
        _convert_body,
        grid=(rows // blk,),
        in_specs=[pl.BlockSpec((blk, EMB), lambda i: (i, 0))],
        out_specs=pl.BlockSpec((blk, EMB), lambda i: (i, 0)),
        out_shape=jax.ShapeDtypeStruct((rows, EMB), jnp.bfloat16),
    )(table)


def _sim_body(q_ref, c_ref, o_ref):
    q = q_ref[...]
    c = c_ref[...]
    qn = q / (jnp.sqrt(jnp.sum(q * q, axis=1, keepdims=True)) + SMALL)
    cn = c / (jnp.sqrt(jnp.sum(c * c, axis=1, keepdims=True)) + SMALL)
    o_ref[...] = lax.dot_general(
        qn, cn, (((1,), (1,)), ((), ())),
        preferred_element_type=jnp.float32)


def _similarity(pooled_q, pooled_c):
    bq = 512
    bc = 512
    return pl.pallas_call(
        _sim_body,
        grid=(B // bq, B // bc),
        in_specs=[
            pl.BlockSpec((bq, EMB), lambda i, j: (i, 0)),
            pl.BlockSpec((bc, EMB), lambda i, j: (j, 0)),
        ],
        out_specs=pl.BlockSpec((bq, bc), lambda i, j: (i, j)),
        out_shape=jax.ShapeDtypeStruct((B, B), jnp.float32),
    )(pooled_q, pooled_c)


def kernel(code_seqs, query_seqs, code_table, query_table):
    code_idx = jnp.pad(code_seqs.astype(jnp.int32), ((0, 0), (0, LCP - LC)))
    query_idx = jnp.pad(query_seqs.astype(jnp.int32), ((0, 0), (0, LQP - LQ)))
    mesh = plsc.VectorSubcoreMesh(core_axis_name="c", subcore_axis_name="s")
    nw = mesh.num_cores * mesh.num_subcores
    pooled_c, pooled_q = _pool_kernel(
        B // nw, code_idx, query_idx,
        code_table.astype(jnp.bfloat16), query_table.astype(jnp.bfloat16))
    return _similarity(pooled_q, pooled_c)


# speedup vs baseline: 1.6066x; 1.6066x over previous
import functools

import jax
import jax.numpy as jnp
from jax import lax
from jax.experimental import pallas as pl
from jax.experimental.pallas import tpu as pltpu
from jax.experimental.pallas import tpu_sc as plsc

SMALL = 1e-10
EMB = 128
B = 4096
LC = 200
LC1 = 104
LQ = 20
NE = EMB // 16


def _pool_kernel(b_per_w, code_idx, query_idx, code_table, query_table):
    mesh = plsc.VectorSubcoreMesh(core_axis_name="c", subcore_axis_name="s")
    nc = mesh.num_cores
    npair = b_per_w // 2

    @functools.partial(
        pl.kernel,
        out_type=[
            jax.ShapeDtypeStruct((B, EMB), jnp.float32),
            jax.ShapeDtypeStruct((B, EMB), jnp.float32),
        ],
        mesh=mesh,
        compiler_params=pltpu.CompilerParams(use_tc_tiling_on_sc=False,
                                             needs_layout_passes=False),
        scratch_types=[
            pltpu.VMEM((b_per_w, LC), jnp.int32),
            pltpu.VMEM((b_per_w, LQ), jnp.int32),
            pltpu.VMEM((LC, EMB), jnp.bfloat16),
            pltpu.VMEM((LC, EMB), jnp.bfloat16),
            pltpu.VMEM((b_per_w, EMB), jnp.float32),
            pltpu.VMEM((b_per_w, EMB), jnp.float32),
            pltpu.SemaphoreType.DMA,
            pltpu.SemaphoreType.DMA,
        ],
    )
    def k(cidx_hbm, qidx_hbm, ctab_hbm, qtab_hbm, outc_hbm, outq_hbm,
          idx_c, idx_q, rows0, rows1, outc, outq, sem0, sem1):
        wid = lax.axis_index("s") * nc + lax.axis_index("c")
        base = wid * b_per_w
        pltpu.sync_copy(cidx_hbm.at[pl.ds(base, b_per_w)], idx_c)
        pltpu.sync_copy(qidx_hbm.at[pl.ds(base, b_per_w)], idx_q)

        def cstart(b, buf, sem):
            pltpu.make_async_copy(
                ctab_hbm.at[idx_c.at[b, pl.ds(0, LC1)]],
                buf.at[pl.ds(0, LC1)], sem).start()
            pltpu.make_async_copy(
                ctab_hbm.at[idx_c.at[b, pl.ds(LC1, LC - LC1)]],
                buf.at[pl.ds(LC1, LC - LC1)], sem).start()

        def cwait(buf, sem):
            pltpu.make_async_copy(
                ctab_hbm.at[idx_c.at[0, pl.ds(0, LC1)]],
                buf.at[pl.ds(0, LC1)], sem).wait()
            pltpu.make_async_copy(
                ctab_hbm.at[idx_c.at[0, pl.ds(LC1, LC - LC1)]],
                buf.at[pl.ds(LC1, LC - LC1)], sem).wait()

        def accum(buf, n, out_ref, b):
            def acc1(l, accs):
                out = list(accs)
                for kk in range(NE // 2):
                    x = buf[l, pl.ds(kk * 32, 32)]
                    u0, u1 = plsc.unpack(
                        x, format=plsc.PackFormat.INTERLEAVED,
                        preferred_element_type=jnp.float32)
                    out[2 * kk] = out[2 * kk] + u0
                    out[2 * kk + 1] = out[2 * kk + 1] + u1
                return tuple(out)
            accs = lax.fori_loop(
                0, n, acc1,
                tuple(jnp.zeros((16,), jnp.float32) for _ in range(NE)))
            for e in range(NE):
                out_ref[b, pl.ds(e * 16, 16)] = accs[e]

        cstart(0, rows0, sem0)

        def code_pair(g, _):
            b0 = 2 * g
            cstart(b0 + 1, rows1, sem1)
            cwait(rows0, sem0)
            accum(rows0, LC, outc, b0)

            @pl.when(g < npair - 1)
            def _():
                cstart(b0 + 2, rows0, sem0)
            cwait(rows1, sem1)
            accum(rows1, LC, outc, b0 + 1)
            return 0

        lax.fori_loop(0, npair, code_pair, 0)

        def qstart(b, buf, sem):
            pltpu.make_async_copy(
                qtab_hbm.at[idx_q.at[b, pl.ds(0, LQ)]],
                buf.at[pl.ds(0, LQ)], sem).start()

        def qwait(buf, sem):
            pltpu.make_async_copy(
                qtab_hbm.at[idx_q.at[0, pl.ds(0, LQ)]],
                buf.at[pl.ds(0, LQ)], sem).wait()

        qstart(0, rows0, sem0)

        def query_pair(g, _):
            b0 = 2 * g
            qstart(b0 + 1, rows1, sem1)
            qwait(rows0, sem0)
            accum(rows0, LQ, outq, b0)

            @pl.when(g < npair - 1)
            def _():
                qstart(b0 + 2, rows0, sem0)
            qwait(rows1, sem1)
            accum(rows1, LQ, outq, b0 + 1)
            return 0

        lax.fori_loop(0, npair, query_pair, 0)

        pltpu.sync_copy(outc, outc_hbm.at[pl.ds(base, b_per_w)])
        pltpu.sync_copy(outq, outq_hbm.at[pl.ds(base, b_per_w)])

    return k(code_idx, query_idx, code_table, query_table)


def _convert_body(x_ref, o_ref):
    o_ref[...] = x_ref[...].astype(jnp.bfloat16)


def _to_bf16(table):
    rows = table.shape[0]
    blk = 5000
    return pl.pallas_call(
        _convert_body,
        grid=(rows // blk,),
        in_specs=[pl.BlockSpec((blk, EMB), lambda i: (i, 0))],
        out_specs=pl.BlockSpec((blk, EMB), lambda i: (i, 0)),
        out_shape=jax.ShapeDtypeStruct((rows, EMB), jnp.bfloat16),
    )(table)


def _convert2_body(c_ref, q_ref, oc_ref, oq_ref):
    oc_ref[...] = c_ref[...].astype(jnp.bfloat16)
    oq_ref[...] = q_ref[...].astype(jnp.bfloat16)


def _both_to_bf16(code_table, query_table):
    rows = code_table.shape[0]
    blk = 5000
    return pl.pallas_call(
        _convert2_body,
        grid=(rows // blk,),
        in_specs=[pl.BlockSpec((blk, EMB), lambda i: (i, 0)),
                  pl.BlockSpec((blk, EMB), lambda i: (i, 0))],
        out_specs=[pl.BlockSpec((blk, EMB), lambda i: (i, 0)),
                   pl.BlockSpec((blk, EMB), lambda i: (i, 0))],
        out_shape=[jax.ShapeDtypeStruct((rows, EMB), jnp.bfloat16),
                   jax.ShapeDtypeStruct((rows, EMB), jnp.bfloat16)],
    )(code_table, query_table)


def _sim_body(q_ref, c_ref, o_ref):
    q = q_ref[...]
    c = c_ref[...]
    qn = q / (jnp.sqrt(jnp.sum(q * q, axis=1, keepdims=True)) + SMALL)
    cn = c / (jnp.sqrt(jnp.sum(c * c, axis=1, keepdims=True)) + SMALL)
    o_ref[...] = lax.dot_general(
        qn.astype(jnp.bfloat16), cn.astype(jnp.bfloat16),
        (((1,), (1,)), ((), ())),
        preferred_element_type=jnp.float32)


def _similarity(pooled_q, pooled_c):
    bq = 512
    bc = 512
    return pl.pallas_call(
        _sim_body,
        grid=(B // bq, B // bc),
        in_specs=[
            pl.BlockSpec((bq, EMB), lambda i, j: (i, 0)),
            pl.BlockSpec((bc, EMB), lambda i, j: (j, 0)),
        ],
        out_specs=pl.BlockSpec((bq, bc), lambda i, j: (i, j)),
        out_shape=jax.ShapeDtypeStruct((B, B), jnp.float32),
    )(pooled_q, pooled_c)


def kernel(code_seqs, query_seqs, code_table, query_table):
    code_idx = code_seqs.astype(jnp.int32)
    query_idx = query_seqs.astype(jnp.int32)
    mesh = plsc.VectorSubcoreMesh(core_axis_name="c", subcore_axis_name="s")
    nw = mesh.num_cores * mesh.num_subcores
    ctab16, qtab16 = _both_to_bf16(code_table, query_table)
    pooled_c, pooled_q = _pool_kernel(
        B // nw, code_idx, query_idx, ctab16, qtab16)
    return _similarity(pooled_q, pooled_c)
